# baseline (speedup 1.0000x reference)
import functools

import jax
import jax.numpy as jnp
from jax import lax
from jax.experimental import pallas as pl
from jax.experimental.pallas import tpu as pltpu
from jax.experimental.pallas import tpu_sc as plsc

NC = 2
NS = 16
NW = NC * NS
CHUNK = 128


@functools.cache
def _build(n_chunks, D):
    mesh = plsc.VectorSubcoreMesh(core_axis_name="c", subcore_axis_name="s")

    @functools.partial(
        pl.kernel,
        out_type=jax.ShapeDtypeStruct((NW, n_chunks, CHUNK, D), jnp.float32),
        mesh=mesh,
        scratch_types=[
            pltpu.VMEM((n_chunks, CHUNK), jnp.int32),
            pltpu.VMEM((CHUNK, D), jnp.float32),
            pltpu.SemaphoreType.DMA,
        ],
        compiler_params=pltpu.CompilerParams(use_tc_tiling_on_sc=False),
    )
    def emb_kernel(idx_hbm, table_hbm, out_hbm, idx_v, rows_v, gsem):
        wid = lax.axis_index("s") * NC + lax.axis_index("c")
        pltpu.sync_copy(idx_hbm.at[wid], idx_v)

        @pl.loop(0, n_chunks)
        def _(j):
            pltpu.async_copy(table_hbm.at[idx_v.at[j]], rows_v, gsem).wait()
            pltpu.sync_copy(rows_v, out_hbm.at[wid, j])

    return emb_kernel


def kernel(x, W):
    B, H = x.shape
    V, D = W.shape
    n_flat = B * H
    assert n_flat % (NW * CHUNK) == 0
    n_chunks = n_flat // (NW * CHUNK)
    idx = x.reshape(NW, n_chunks, CHUNK)
    out = _build(n_chunks, D)(idx, W)
    return out.reshape(B, H * D)


# speedup vs baseline: 1.0719x; 1.0719x over previous
import functools

import jax
import jax.numpy as jnp
from jax import lax
from jax.experimental import pallas as pl
from jax.experimental.pallas import tpu as pltpu
from jax.experimental.pallas import tpu_sc as plsc

NC = 2
NS = 16
NW = NC * NS
CHUNK = 128
K = 5


@functools.cache
def _build(n_chunks, D):
    n_groups = n_chunks // K
    assert n_groups % 2 == 0 and n_groups >= 4
    mesh = plsc.VectorSubcoreMesh(core_axis_name="c", subcore_axis_name="s")

    @functools.partial(
        pl.kernel,
        out_type=jax.ShapeDtypeStruct((NW, n_chunks, CHUNK, D), jnp.float32),
        mesh=mesh,
        scratch_types=[
            pltpu.VMEM((n_chunks, CHUNK), jnp.int32),
            pltpu.VMEM((2, K, CHUNK, D), jnp.float32),
            pltpu.SemaphoreType.DMA,
            pltpu.SemaphoreType.DMA,
            pltpu.SemaphoreType.DMA,
            pltpu.SemaphoreType.DMA,
        ],
        compiler_params=pltpu.CompilerParams(use_tc_tiling_on_sc=False),
    )
    def emb_kernel(idx_hbm, table_hbm, out_hbm, idx_v, rows_v,
                   gsem_a, gsem_b, ssem_a, ssem_b):
        wid = lax.axis_index("s") * NC + lax.axis_index("c")
        pltpu.sync_copy(idx_hbm.at[wid], idx_v)

        def fire_gathers(g, half, sem):
            for b in range(K):
                pltpu.async_copy(
                    table_hbm.at[idx_v.at[g * K + b]], rows_v.at[half, b], sem)

        def drain_gathers(half, sem):
            for b in range(K):
                pltpu.make_async_copy(
                    table_hbm.at[idx_v.at[0]], rows_v.at[half, b], sem).wait()

        def fire_stores(g, half, sem):
            for b in range(K):
                pltpu.async_copy(
                    rows_v.at[half, b], out_hbm.at[wid, g * K + b], sem)

        def drain_stores(half, sem):
            for b in range(K):
                pltpu.make_async_copy(
                    rows_v.at[half, b], out_hbm.at[wid, b], sem).wait()

        fire_gathers(0, 0, gsem_a)
        drain_gathers(0, gsem_a)
        fire_gathers(1, 1, gsem_b)
        fire_stores(0, 0, ssem_a)

        @pl.loop(1, n_groups - 1, step=2)
        def _(g):
            drain_gathers(1, gsem_b)
            drain_stores(0, ssem_a)
            fire_gathers(g + 1, 0, gsem_a)
            fire_stores(g, 1, ssem_b)

            drain_gathers(0, gsem_a)
            drain_stores(1, ssem_b)
            fire_gathers(g + 2, 1, gsem_b)
            fire_stores(g + 1, 0, ssem_a)

        drain_gathers(1, gsem_b)
        drain_stores(0, ssem_a)
        fire_stores(n_groups - 1, 1, ssem_b)
        drain_stores(1, ssem_b)

    return emb_kernel


def kernel(x, W):
    B, H = x.shape
    V, D = W.shape
    n_flat = B * H
    assert n_flat % (NW * CHUNK) == 0
    n_chunks = n_flat // (NW * CHUNK)
    idx = x.reshape(NW, n_chunks, CHUNK)
    out = _build(n_chunks, D)(idx, W)
    return out.reshape(B, H * D)
